# Initial kernel scaffold; baseline (speedup 1.0000x reference)
import jax
import jax.numpy as jnp
from jax.experimental import pallas as pl


def kernel(X, steps, mask, emb_table, gamma, beta, noise):
    raise NotImplementedError("write your pallas kernel here")


# speedup vs baseline: 2.5713x; 2.5713x over previous
import functools
import math

import jax
import jax.numpy as jnp
import numpy as np
from jax import lax
from jax.experimental import pallas as pl
from jax.experimental.pallas import tpu as pltpu
from jax.experimental.pallas import tpu_sc as plsc

MAX_STEPS = 1000
PE_MAX = 512


def _sin_cos_encoding(d_model, max_len):
    pe = np.zeros((max_len, d_model), dtype=np.float32)
    position = np.arange(0, max_len, dtype=np.float32)[:, None]
    div_term = np.exp(
        np.arange(0, d_model, 2, dtype=np.float32) * -(math.log(10000.0) / d_model)
    )
    pe[:, 0::2] = np.sin(position * div_term)
    pe[:, 1::2] = np.cos(position * div_term)
    return pe


def _make_sc_gather(V, D, N, B, chunk):
    info = plsc.get_sparse_core_info()
    nw = info.num_cores * info.num_subcores
    assert N % (nw * chunk) == 0 and B % nw == 0
    per_w = N // nw
    nchunks = per_w // chunk
    b_per_w = B // nw
    mesh = plsc.VectorSubcoreMesh(core_axis_name="c", subcore_axis_name="s")

    @functools.partial(
        pl.kernel,
        mesh=mesh,
        out_type=(
            jax.ShapeDtypeStruct((N, D), jnp.float32),
            jax.ShapeDtypeStruct((B, D), jnp.float32),
        ),
        scratch_types=[
            pltpu.VMEM((chunk,), jnp.int32),
            pltpu.VMEM((chunk, D), jnp.float32),
            pltpu.VMEM((b_per_w,), jnp.int32),
            pltpu.VMEM((b_per_w, D), jnp.float32),
            pltpu.SemaphoreType.DMA,
        ],
    )
    def sc_gather(table_hbm, idx_hbm, pestep_hbm, steps_hbm,
                  emb_out, steprow_out, idx_v, rows_v, sidx_v, srows_v, sem):
        wid = lax.axis_index("s") * info.num_cores + lax.axis_index("c")

        sbase = wid * b_per_w
        pltpu.sync_copy(steps_hbm.at[pl.ds(sbase, b_per_w)], sidx_v)
        pltpu.async_copy(pestep_hbm.at[sidx_v], srows_v, sem).wait()
        pltpu.sync_copy(srows_v, steprow_out.at[pl.ds(sbase, b_per_w)])

        def body(j, carry):
            base = wid * per_w + j * chunk
            pltpu.sync_copy(idx_hbm.at[pl.ds(base, chunk)], idx_v)
            pltpu.async_copy(table_hbm.at[idx_v], rows_v, sem).wait()
            pltpu.sync_copy(rows_v, emb_out.at[pl.ds(base, chunk)])
            return carry

        lax.fori_loop(0, nchunks, body, 0)

    return sc_gather


def _epilogue_body(emb_ref, noise_ref, mask_ref, steps_ref, steprow_ref,
                   pe_ref, gamma_ref, beta_ref, out_ref):
    e = emb_ref[...]
    nz = noise_ref[...]
    m = mask_ref[...]
    stf = steps_ref[...]
    ni = 1.0 - jnp.cos(jnp.pi * (1.0 - stf * (1.0 / MAX_STEPS)) * 0.5)
    x = e + m * ((ni - 1.0) * e + (1.0 - ni) * nz)
    x = x + pe_ref[...]
    x = x + m * steprow_ref[...]
    mean = jnp.mean(x, axis=-1, keepdims=True)
    xc = x - mean
    var = jnp.mean(xc * xc, axis=-1, keepdims=True)
    out_ref[...] = xc * lax.rsqrt(var + 1e-5) * gamma_ref[...] + beta_ref[...]


def _epilogue(embed, noise, maskf, stepsf, steprows, pe_pos, gamma3, beta3, BB):
    B, S, D = embed.shape
    grid = (B // BB,)
    bspec = pl.BlockSpec((BB, S, D), lambda i: (i, 0, 0))
    return pl.pallas_call(
        _epilogue_body,
        grid=grid,
        in_specs=[
            bspec,
            bspec,
            pl.BlockSpec((BB, S, 1), lambda i: (i, 0, 0)),
            pl.BlockSpec((BB, 1, 1), lambda i: (i, 0, 0)),
            pl.BlockSpec((BB, 1, D), lambda i: (i, 0, 0)),
            pl.BlockSpec((1, S, D), lambda i: (0, 0, 0)),
            pl.BlockSpec((1, 1, D), lambda i: (0, 0, 0)),
            pl.BlockSpec((1, 1, D), lambda i: (0, 0, 0)),
        ],
        out_specs=bspec,
        out_shape=jax.ShapeDtypeStruct((B, S, D), jnp.float32),
        compiler_params=pltpu.CompilerParams(
            dimension_semantics=("parallel",),
        ),
    )(embed, noise, maskf, stepsf, steprows, pe_pos, gamma3, beta3)


def kernel(X, steps, mask, emb_table, gamma, beta, noise):
    B, S = X.shape
    V, D = emb_table.shape
    N = B * S

    idx = X.reshape(N).astype(jnp.int32)
    steps_i = steps.astype(jnp.int32)
    pe_step = jnp.asarray(_sin_cos_encoding(D, MAX_STEPS))
    pe_pos = jnp.asarray(_sin_cos_encoding(D, PE_MAX)[:S][None])

    sc = _make_sc_gather(V, D, N, B, chunk=128)
    embed_flat, steprows = sc(emb_table, idx, pe_step, steps_i)

    embed = embed_flat.reshape(B, S, D)
    maskf = mask.astype(jnp.float32)[:, :, None]
    stepsf = steps.astype(jnp.float32).reshape(B, 1, 1)
    steprows3 = steprows.reshape(B, 1, D)
    out = _epilogue(embed, noise, maskf, stepsf, steprows3,
                    pe_pos, gamma.reshape(1, 1, D), beta.reshape(1, 1, D), BB=8)
    return out


# speedup vs baseline: 2.8780x; 1.1193x over previous
import functools
import math

import jax
import jax.numpy as jnp
import numpy as np
from jax import lax
from jax.experimental import pallas as pl
from jax.experimental.pallas import tpu as pltpu
from jax.experimental.pallas import tpu_sc as plsc

MAX_STEPS = 1000
PE_MAX = 512


def _sin_cos_encoding(d_model, max_len):
    pe = np.zeros((max_len, d_model), dtype=np.float32)
    position = np.arange(0, max_len, dtype=np.float32)[:, None]
    div_term = np.exp(
        np.arange(0, d_model, 2, dtype=np.float32) * -(math.log(10000.0) / d_model)
    )
    pe[:, 0::2] = np.sin(position * div_term)
    pe[:, 1::2] = np.cos(position * div_term)
    return pe


def _make_sc_gather(V, D, N, B, chunk):
    info = plsc.get_sparse_core_info()
    nw = info.num_cores * info.num_subcores
    assert N % (nw * chunk) == 0 and B % nw == 0
    per_w = N // nw
    nchunks = per_w // chunk
    b_per_w = B // nw
    mesh = plsc.VectorSubcoreMesh(core_axis_name="c", subcore_axis_name="s")

    NBUF = 5
    assert nchunks % NBUF == 0 and nchunks >= NBUF

    @functools.partial(
        pl.kernel,
        mesh=mesh,
        out_type=(
            jax.ShapeDtypeStruct((N, D), jnp.float32),
            jax.ShapeDtypeStruct((B, D), jnp.float32),
        ),
        scratch_types=(
            [pltpu.VMEM((nchunks, chunk), jnp.int32)]
            + [pltpu.VMEM((chunk, D), jnp.float32) for _ in range(NBUF)]
            + [pltpu.VMEM((b_per_w,), jnp.int32),
               pltpu.VMEM((b_per_w, D), jnp.float32)]
            + [pltpu.SemaphoreType.DMA for _ in range(2 * NBUF + 1)]
        ),
    )
    def sc_gather(table_hbm, idx_hbm, pestep_hbm, steps_hbm,
                  emb_out, steprow_out, idx_v, *rest):
        rows = rest[:NBUF]
        sidx_v, srows_v = rest[NBUF:NBUF + 2]
        gsem = rest[NBUF + 2:2 * NBUF + 2]
        wsem = rest[2 * NBUF + 2:3 * NBUF + 2]
        sem = rest[3 * NBUF + 2]
        wid = lax.axis_index("s") * info.num_cores + lax.axis_index("c")

        def g_start(k, r):
            pltpu.async_copy(table_hbm.at[idx_v.at[k]], rows[r], gsem[r])

        def g_wait(k, r):
            pltpu.make_async_copy(table_hbm.at[idx_v.at[k]], rows[r], gsem[r]).wait()

        def w_start(k, r):
            base = wid * per_w + k * chunk
            pltpu.async_copy(rows[r], emb_out.at[pl.ds(base, chunk)], wsem[r])

        def w_wait(k, r):
            base = wid * per_w + k * chunk
            pltpu.make_async_copy(rows[r], emb_out.at[pl.ds(base, chunk)], wsem[r]).wait()

        pltpu.sync_copy(idx_hbm.at[wid], idx_v)
        g_start(0, 0)
        g_start(1, 1)
        g_start(2, 2)

        sbase = wid * b_per_w
        pltpu.sync_copy(steps_hbm.at[pl.ds(sbase, b_per_w)], sidx_v)
        pltpu.async_copy(pestep_hbm.at[sidx_v], srows_v, sem).wait()
        pltpu.sync_copy(srows_v, steprow_out.at[pl.ds(sbase, b_per_w)])

        def body(t, carry):
            for b in range(NBUF):
                k = NBUF * t + b
                g_wait(k, b)
                w_start(k, b)
                bj = (b + 3) % NBUF

                @pl.when(k + 3 >= NBUF)
                def _():
                    w_wait(k + 3 - NBUF, bj)

                @pl.when(k + 3 < nchunks)
                def _():
                    g_start(k + 3, bj)

            return carry

        lax.fori_loop(0, nchunks // NBUF, body, 0)
        for k in range(nchunks - (NBUF - 3), nchunks):
            w_wait(k, k % NBUF)

    return sc_gather


def _epilogue_body(emb_ref, noise_ref, mask_ref, steps_ref, steprow_ref,
                   pe_ref, gamma_ref, beta_ref, out_ref):
    e = emb_ref[...]
    nz = noise_ref[...]
    m = mask_ref[...]
    stf = steps_ref[...]
    ni = 1.0 - jnp.cos(jnp.pi * (1.0 - stf * (1.0 / MAX_STEPS)) * 0.5)
    x = e + m * ((ni - 1.0) * e + (1.0 - ni) * nz)
    x = x + pe_ref[...]
    x = x + m * steprow_ref[...]
    mean = jnp.mean(x, axis=-1, keepdims=True)
    xc = x - mean
    var = jnp.mean(xc * xc, axis=-1, keepdims=True)
    out_ref[...] = xc * lax.rsqrt(var + 1e-5) * gamma_ref[...] + beta_ref[...]


def _epilogue(embed, noise, maskf, stepsf, steprows, pe_pos, gamma3, beta3, BB):
    B, S, D = embed.shape
    grid = (B // BB,)
    bspec = pl.BlockSpec((BB, S, D), lambda i: (i, 0, 0))
    return pl.pallas_call(
        _epilogue_body,
        grid=grid,
        in_specs=[
            bspec,
            bspec,
            pl.BlockSpec((BB, S, 1), lambda i: (i, 0, 0)),
            pl.BlockSpec((BB, 1, 1), lambda i: (i, 0, 0)),
            pl.BlockSpec((BB, 1, D), lambda i: (i, 0, 0)),
            pl.BlockSpec((1, S, D), lambda i: (0, 0, 0)),
            pl.BlockSpec((1, 1, D), lambda i: (0, 0, 0)),
            pl.BlockSpec((1, 1, D), lambda i: (0, 0, 0)),
        ],
        out_specs=bspec,
        out_shape=jax.ShapeDtypeStruct((B, S, D), jnp.float32),
        compiler_params=pltpu.CompilerParams(
            dimension_semantics=("parallel",),
        ),
    )(embed, noise, maskf, stepsf, steprows, pe_pos, gamma3, beta3)


def kernel(X, steps, mask, emb_table, gamma, beta, noise):
    B, S = X.shape
    V, D = emb_table.shape
    N = B * S

    chunk = 128
    nw = 32
    idx = X.reshape(nw, N // (nw * chunk), chunk).astype(jnp.int32)
    steps_i = steps.astype(jnp.int32)
    pe_step = jnp.asarray(_sin_cos_encoding(D, MAX_STEPS))
    pe_pos = jnp.asarray(_sin_cos_encoding(D, PE_MAX)[:S][None])

    sc = _make_sc_gather(V, D, N, B, chunk=chunk)
    embed_flat, steprows = sc(emb_table, idx, pe_step, steps_i)

    embed = embed_flat.reshape(B, S, D)
    maskf = mask.astype(jnp.float32)[:, :, None]
    stepsf = steps.astype(jnp.float32).reshape(B, 1, 1)
    steprows3 = steprows.reshape(B, 1, D)
    out = _epilogue(embed, noise, maskf, stepsf, steprows3,
                    pe_pos, gamma.reshape(1, 1, D), beta.reshape(1, 1, D), BB=8)
    return out


# speedup vs baseline: 3.4335x; 1.1930x over previous
import functools
import math

import jax
import jax.numpy as jnp
import numpy as np
from jax import lax
from jax.experimental import pallas as pl
from jax.experimental.pallas import tpu as pltpu
from jax.experimental.pallas import tpu_sc as plsc

MAX_STEPS = 1000
PE_MAX = 512


def _sin_cos_encoding(d_model, max_len):
    pe = np.zeros((max_len, d_model), dtype=np.float32)
    position = np.arange(0, max_len, dtype=np.float32)[:, None]
    div_term = np.exp(
        np.arange(0, d_model, 2, dtype=np.float32) * -(math.log(10000.0) / d_model)
    )
    pe[:, 0::2] = np.sin(position * div_term)
    pe[:, 1::2] = np.cos(position * div_term)
    return pe


def _make_sc_gather(V, D, N, B, chunk):
    info = plsc.get_sparse_core_info()
    nw = info.num_cores * info.num_subcores
    assert N % (nw * chunk) == 0 and B % nw == 0
    per_w = N // nw
    nchunks = per_w // chunk
    b_per_w = B // nw
    mesh = plsc.VectorSubcoreMesh(core_axis_name="c", subcore_axis_name="s")

    NBUF = 5
    assert nchunks % NBUF == 0 and nchunks >= NBUF

    @functools.partial(
        pl.kernel,
        mesh=mesh,
        out_type=(
            jax.ShapeDtypeStruct((N, D), jnp.float32),
            jax.ShapeDtypeStruct((B, D), jnp.float32),
        ),
        scratch_types=(
            [pltpu.VMEM((nchunks, chunk), jnp.int32)]
            + [pltpu.VMEM((chunk, D), jnp.float32) for _ in range(NBUF)]
            + [pltpu.VMEM((b_per_w,), jnp.int32),
               pltpu.VMEM((b_per_w, D), jnp.float32)]
            + [pltpu.SemaphoreType.DMA for _ in range(2 * NBUF + 1)]
        ),
    )
    def sc_gather(table_hbm, idx_hbm, pestep_hbm, steps_hbm,
                  emb_out, steprow_out, idx_v, *rest):
        rows = rest[:NBUF]
        sidx_v, srows_v = rest[NBUF:NBUF + 2]
        gsem = rest[NBUF + 2:2 * NBUF + 2]
        wsem = rest[2 * NBUF + 2:3 * NBUF + 2]
        sem = rest[3 * NBUF + 2]
        wid = lax.axis_index("s") * info.num_cores + lax.axis_index("c")

        def g_start(k, r):
            pltpu.async_copy(table_hbm.at[idx_v.at[k]], rows[r], gsem[r])

        def g_wait(k, r):
            pltpu.make_async_copy(table_hbm.at[idx_v.at[k]], rows[r], gsem[r]).wait()

        def w_start(k, r):
            base = wid * per_w + k * chunk
            pltpu.async_copy(rows[r], emb_out.at[pl.ds(base, chunk)], wsem[r])

        def w_wait(k, r):
            base = wid * per_w + k * chunk
            pltpu.make_async_copy(rows[r], emb_out.at[pl.ds(base, chunk)], wsem[r]).wait()

        pltpu.sync_copy(idx_hbm.at[wid], idx_v)
        g_start(0, 0)
        g_start(1, 1)
        g_start(2, 2)

        sbase = wid * b_per_w
        pltpu.sync_copy(steps_hbm.at[pl.ds(sbase, b_per_w)], sidx_v)
        pltpu.async_copy(pestep_hbm.at[sidx_v], srows_v, sem).wait()
        pltpu.sync_copy(srows_v, steprow_out.at[pl.ds(sbase, b_per_w)])

        def body(t, carry):
            for b in range(NBUF):
                k = NBUF * t + b
                g_wait(k, b)
                w_start(k, b)
                bj = (b + 3) % NBUF

                @pl.when(k + 3 >= NBUF)
                def _():
                    w_wait(k + 3 - NBUF, bj)

                @pl.when(k + 3 < nchunks)
                def _():
                    g_start(k + 3, bj)

            return carry

        lax.fori_loop(0, nchunks // NBUF, body, 0)
        for k in range(nchunks - (NBUF - 3), nchunks):
            w_wait(k, k % NBUF)

    return sc_gather


def _epilogue_body(emb_ref, noise_ref, mask_ref, steps_ref, steprow_ref,
                   pe_ref, gamma_ref, beta_ref, out_ref):
    e = emb_ref[...]
    nz = noise_ref[...]
    m = mask_ref[...][:, :, None]
    stf = steps_ref[...]
    ni = 1.0 - jnp.cos(jnp.pi * (1.0 - stf * (1.0 / MAX_STEPS)) * 0.5)
    x = e + m * ((ni - 1.0) * e + (1.0 - ni) * nz)
    x = x + pe_ref[...]
    x = x + m * steprow_ref[...]
    mean = jnp.mean(x, axis=-1, keepdims=True)
    xc = x - mean
    var = jnp.mean(xc * xc, axis=-1, keepdims=True)
    out_ref[...] = xc * lax.rsqrt(var + 1e-5) * gamma_ref[...] + beta_ref[...]


def _epilogue(embed, noise, maskf, stepsf, steprows, pe_pos, gamma3, beta3, BB):
    B, S, D = embed.shape
    grid = (B // BB,)
    bspec = pl.BlockSpec((BB, S, D), lambda i: (i, 0, 0))
    return pl.pallas_call(
        _epilogue_body,
        grid=grid,
        in_specs=[
            bspec,
            bspec,
            pl.BlockSpec((BB, S), lambda i: (i, 0)),
            pl.BlockSpec((BB, 1, 1), lambda i: (i, 0, 0)),
            pl.BlockSpec((BB, 1, D), lambda i: (i, 0, 0)),
            pl.BlockSpec((1, S, D), lambda i: (0, 0, 0)),
            pl.BlockSpec((1, 1, D), lambda i: (0, 0, 0)),
            pl.BlockSpec((1, 1, D), lambda i: (0, 0, 0)),
        ],
        out_specs=bspec,
        out_shape=jax.ShapeDtypeStruct((B, S, D), jnp.float32),
        compiler_params=pltpu.CompilerParams(
            dimension_semantics=("parallel",),
        ),
    )(embed, noise, maskf, stepsf, steprows, pe_pos, gamma3, beta3)


def kernel(X, steps, mask, emb_table, gamma, beta, noise):
    B, S = X.shape
    V, D = emb_table.shape
    N = B * S

    chunk = 128
    nw = 32
    idx = X.reshape(nw, N // (nw * chunk), chunk).astype(jnp.int32)
    steps_i = steps.astype(jnp.int32)
    pe_step = jnp.asarray(_sin_cos_encoding(D, MAX_STEPS))
    pe_pos = jnp.asarray(_sin_cos_encoding(D, PE_MAX)[:S][None])

    sc = _make_sc_gather(V, D, N, B, chunk=chunk)
    embed_flat, steprows = sc(emb_table, idx, pe_step, steps_i)

    embed = embed_flat.reshape(B, S, D)
    maskf = mask.astype(jnp.float32)
    stepsf = steps.astype(jnp.float32).reshape(B, 1, 1)
    steprows3 = steprows.reshape(B, 1, D)
    out = _epilogue(embed, noise, maskf, stepsf, steprows3,
                    pe_pos, gamma.reshape(1, 1, D), beta.reshape(1, 1, D), BB=8)
    return out


# speedup vs baseline: 3.9567x; 1.1524x over previous
import functools
import math

import jax
import jax.numpy as jnp
import numpy as np
from jax import lax
from jax.experimental import pallas as pl
from jax.experimental.pallas import tpu as pltpu
from jax.experimental.pallas import tpu_sc as plsc

MAX_STEPS = 1000
PE_MAX = 512


def _sin_cos_encoding(d_model, max_len):
    pe = np.zeros((max_len, d_model), dtype=np.float32)
    position = np.arange(0, max_len, dtype=np.float32)[:, None]
    div_term = np.exp(
        np.arange(0, d_model, 2, dtype=np.float32) * -(math.log(10000.0) / d_model)
    )
    pe[:, 0::2] = np.sin(position * div_term)
    pe[:, 1::2] = np.cos(position * div_term)
    return pe


def _make_sc_gather(V, D, N, B, chunk):
    info = plsc.get_sparse_core_info()
    nw = info.num_cores * info.num_subcores
    assert N % (nw * chunk) == 0 and B % nw == 0
    per_w = N // nw
    nchunks = per_w // chunk
    b_per_w = B // nw
    mesh = plsc.VectorSubcoreMesh(core_axis_name="c", subcore_axis_name="s")

    NBUF = 5
    assert nchunks % NBUF == 0 and nchunks >= NBUF

    @functools.partial(
        pl.kernel,
        mesh=mesh,
        out_type=(
            jax.ShapeDtypeStruct((N, D), jnp.float32),
            jax.ShapeDtypeStruct((B, D), jnp.float32),
        ),
        scratch_types=(
            [pltpu.VMEM((nchunks, chunk), jnp.int32)]
            + [pltpu.VMEM((chunk, D), jnp.float32) for _ in range(NBUF)]
            + [pltpu.VMEM((b_per_w,), jnp.int32),
               pltpu.VMEM((b_per_w, D), jnp.float32)]
            + [pltpu.SemaphoreType.DMA for _ in range(2 * NBUF + 1)]
        ),
    )
    def sc_gather(table_hbm, idx_hbm, pestep_hbm, steps_hbm,
                  emb_out, steprow_out, idx_v, *rest):
        rows = rest[:NBUF]
        sidx_v, srows_v = rest[NBUF:NBUF + 2]
        gsem = rest[NBUF + 2:2 * NBUF + 2]
        wsem = rest[2 * NBUF + 2:3 * NBUF + 2]
        sem = rest[3 * NBUF + 2]
        wid = lax.axis_index("s") * info.num_cores + lax.axis_index("c")

        def g_start(k, r):
            pltpu.async_copy(table_hbm.at[idx_v.at[k]], rows[r], gsem[r])

        def g_wait(k, r):
            pltpu.make_async_copy(table_hbm.at[idx_v.at[k]], rows[r], gsem[r]).wait()

        def w_start(k, r):
            base = wid * per_w + k * chunk
            pltpu.async_copy(rows[r], emb_out.at[pl.ds(base, chunk)], wsem[r])

        def w_wait(k, r):
            base = wid * per_w + k * chunk
            pltpu.make_async_copy(rows[r], emb_out.at[pl.ds(base, chunk)], wsem[r]).wait()

        pltpu.sync_copy(idx_hbm.at[wid], idx_v)
        g_start(0, 0)
        g_start(1, 1)
        g_start(2, 2)

        sbase = wid * b_per_w
        pltpu.sync_copy(steps_hbm.at[pl.ds(sbase, b_per_w)], sidx_v)
        pltpu.async_copy(pestep_hbm.at[sidx_v], srows_v, sem).wait()
        pltpu.sync_copy(srows_v, steprow_out.at[pl.ds(sbase, b_per_w)])

        def body(t, carry):
            for b in range(NBUF):
                k = NBUF * t + b
                g_wait(k, b)
                w_start(k, b)
                bj = (b + 3) % NBUF

                @pl.when(k + 3 >= NBUF)
                def _():
                    w_wait(k + 3 - NBUF, bj)

                @pl.when(k + 3 < nchunks)
                def _():
                    g_start(k + 3, bj)

            return carry

        lax.fori_loop(0, nchunks // NBUF, body, 0)
        for k in range(nchunks - (NBUF - 3), nchunks):
            w_wait(k, k % NBUF)

    return sc_gather


def _epilogue_body(emb_ref, noise_ref, mask_ref, steps_ref, steprow_ref,
                   pe_ref, gamma_ref, beta_ref, out_ref):
    e = emb_ref[...]
    nz = noise_ref[...]
    m = mask_ref[...][:, :, None]
    stf = steps_ref[...]
    ni = 1.0 - jnp.cos(jnp.pi * (1.0 - stf * (1.0 / MAX_STEPS)) * 0.5)
    x = e + m * ((ni - 1.0) * e + (1.0 - ni) * nz)
    x = x + pe_ref[...]
    x = x + m * steprow_ref[...]
    mean = jnp.mean(x, axis=-1, keepdims=True)
    xc = x - mean
    var = jnp.mean(xc * xc, axis=-1, keepdims=True)
    out_ref[...] = xc * lax.rsqrt(var + 1e-5) * gamma_ref[...] + beta_ref[...]


def _epilogue(embed, noise, maskf, stepsf, steprows, pe_pos, gamma3, beta3, BB):
    B, S, D = embed.shape
    grid = (B // BB,)
    bspec = pl.BlockSpec((BB, S, D), lambda i: (i, 0, 0))
    return pl.pallas_call(
        _epilogue_body,
        grid=grid,
        in_specs=[
            bspec,
            bspec,
            pl.BlockSpec((BB, S), lambda i: (i, 0)),
            pl.BlockSpec((BB, 1, 1), lambda i: (i, 0, 0)),
            pl.BlockSpec((BB, 1, D), lambda i: (i, 0, 0)),
            pl.BlockSpec((1, S, D), lambda i: (0, 0, 0)),
            pl.BlockSpec((1, 1, D), lambda i: (0, 0, 0)),
            pl.BlockSpec((1, 1, D), lambda i: (0, 0, 0)),
        ],
        out_specs=bspec,
        out_shape=jax.ShapeDtypeStruct((B, S, D), jnp.float32),
        compiler_params=pltpu.CompilerParams(
            dimension_semantics=("parallel",),
        ),
    )(embed, noise, maskf, stepsf, steprows, pe_pos, gamma3, beta3)


def kernel(X, steps, mask, emb_table, gamma, beta, noise):
    B, S = X.shape
    V, D = emb_table.shape
    N = B * S

    chunk = 128
    nw = 32
    idx = X.reshape(nw, N // (nw * chunk), chunk).astype(jnp.int32)
    steps_i = steps.astype(jnp.int32)
    pe_step = jnp.asarray(_sin_cos_encoding(D, MAX_STEPS))
    pe_pos = jnp.asarray(_sin_cos_encoding(D, PE_MAX)[:S][None])

    sc = _make_sc_gather(V, D, N, B, chunk=chunk)
    embed_flat, steprows = sc(emb_table, idx, pe_step, steps_i)

    embed = embed_flat.reshape(B, S, D)
    maskf = mask.astype(jnp.float32)
    stepsf = steps.astype(jnp.float32).reshape(B, 1, 1)
    steprows3 = steprows.reshape(B, 1, D)
    out = _epilogue(embed, noise, maskf, stepsf, steprows3,
                    pe_pos, gamma.reshape(1, 1, D), beta.reshape(1, 1, D), BB=16)
    return out


# speedup vs baseline: 4.1787x; 1.0561x over previous
import functools
import math

import jax
import jax.numpy as jnp
import numpy as np
from jax import lax
from jax.experimental import pallas as pl
from jax.experimental.pallas import tpu as pltpu
from jax.experimental.pallas import tpu_sc as plsc

MAX_STEPS = 1000
PE_MAX = 512


def _sin_cos_encoding(d_model, max_len):
    pe = np.zeros((max_len, d_model), dtype=np.float32)
    position = np.arange(0, max_len, dtype=np.float32)[:, None]
    div_term = np.exp(
        np.arange(0, d_model, 2, dtype=np.float32) * -(math.log(10000.0) / d_model)
    )
    pe[:, 0::2] = np.sin(position * div_term)
    pe[:, 1::2] = np.cos(position * div_term)
    return pe


def _make_sc_gather(V, D, N, B, chunk):
    info = plsc.get_sparse_core_info()
    nw = info.num_cores * info.num_subcores
    assert N % (nw * chunk) == 0 and B % nw == 0
    per_w = N // nw
    nchunks = per_w // chunk
    b_per_w = B // nw
    mesh = plsc.VectorSubcoreMesh(core_axis_name="c", subcore_axis_name="s")

    NBUF = 5
    assert nchunks % NBUF == 0 and nchunks >= NBUF

    @functools.partial(
        pl.kernel,
        mesh=mesh,
        out_type=(
            jax.ShapeDtypeStruct((N, D), jnp.float32),
            jax.ShapeDtypeStruct((B, D), jnp.float32),
        ),
        scratch_types=(
            [pltpu.VMEM((nchunks, chunk), jnp.int32)]
            + [pltpu.VMEM((chunk, D), jnp.float32) for _ in range(NBUF)]
            + [pltpu.VMEM((b_per_w,), jnp.int32),
               pltpu.VMEM((b_per_w, D), jnp.float32)]
            + [pltpu.SemaphoreType.DMA for _ in range(2 * NBUF + 1)]
        ),
    )
    def sc_gather(table_hbm, idx_hbm, pestep_hbm, steps_hbm,
                  emb_out, steprow_out, idx_v, *rest):
        rows = rest[:NBUF]
        sidx_v, srows_v = rest[NBUF:NBUF + 2]
        gsem = rest[NBUF + 2:2 * NBUF + 2]
        wsem = rest[2 * NBUF + 2:3 * NBUF + 2]
        sem = rest[3 * NBUF + 2]
        wid = lax.axis_index("s") * info.num_cores + lax.axis_index("c")

        def g_start(k, r):
            pltpu.async_copy(table_hbm.at[idx_v.at[k]], rows[r], gsem[r])

        def g_wait(k, r):
            pltpu.make_async_copy(table_hbm.at[idx_v.at[k]], rows[r], gsem[r]).wait()

        def w_start(k, r):
            base = wid * per_w + k * chunk
            pltpu.async_copy(rows[r], emb_out.at[pl.ds(base, chunk)], wsem[r])

        def w_wait(k, r):
            base = wid * per_w + k * chunk
            pltpu.make_async_copy(rows[r], emb_out.at[pl.ds(base, chunk)], wsem[r]).wait()

        pltpu.sync_copy(idx_hbm.at[wid], idx_v)
        g_start(0, 0)
        g_start(1, 1)
        g_start(2, 2)

        sbase = wid * b_per_w
        pltpu.sync_copy(steps_hbm.at[pl.ds(sbase, b_per_w)], sidx_v)
        pltpu.async_copy(pestep_hbm.at[sidx_v], srows_v, sem).wait()
        pltpu.sync_copy(srows_v, steprow_out.at[pl.ds(sbase, b_per_w)])

        def body(t, carry):
            for b in range(NBUF):
                k = NBUF * t + b
                g_wait(k, b)
                w_start(k, b)
                bj = (b + 3) % NBUF

                @pl.when(k + 3 >= NBUF)
                def _():
                    w_wait(k + 3 - NBUF, bj)

                @pl.when(k + 3 < nchunks)
                def _():
                    g_start(k + 3, bj)

            return carry

        lax.fori_loop(0, nchunks // NBUF, body, 0)
        for k in range(nchunks - (NBUF - 3), nchunks):
            w_wait(k, k % NBUF)

    return sc_gather


def _epilogue_body(emb_ref, noise_ref, mask_ref, steps_ref, steprow_ref,
                   pe_ref, gamma_ref, beta_ref, out_ref):
    e = emb_ref[...]
    nz = noise_ref[...]
    m = mask_ref[...][:, :, None]
    stf = steps_ref[...]
    ni = 1.0 - jnp.cos(jnp.pi * (1.0 - stf * (1.0 / MAX_STEPS)) * 0.5)
    x = e + m * ((ni - 1.0) * e + (1.0 - ni) * nz)
    x = x + pe_ref[...]
    x = x + m * steprow_ref[...]
    mean = jnp.mean(x, axis=-1, keepdims=True)
    xc = x - mean
    var = jnp.mean(xc * xc, axis=-1, keepdims=True)
    out_ref[...] = xc * lax.rsqrt(var + 1e-5) * gamma_ref[...] + beta_ref[...]


def _epilogue(embed, noise, maskf, stepsf, steprows, pe_pos, gamma3, beta3, BB):
    B, S, D = embed.shape
    grid = (B // BB,)
    bspec = pl.BlockSpec((BB, S, D), lambda i: (i, 0, 0))
    return pl.pallas_call(
        _epilogue_body,
        grid=grid,
        in_specs=[
            bspec,
            bspec,
            pl.BlockSpec((BB, S), lambda i: (i, 0)),
            pl.BlockSpec((BB, 1, 1), lambda i: (i, 0, 0)),
            pl.BlockSpec((BB, 1, D), lambda i: (i, 0, 0)),
            pl.BlockSpec((1, S, D), lambda i: (0, 0, 0)),
            pl.BlockSpec((1, 1, D), lambda i: (0, 0, 0)),
            pl.BlockSpec((1, 1, D), lambda i: (0, 0, 0)),
        ],
        out_specs=bspec,
        out_shape=jax.ShapeDtypeStruct((B, S, D), jnp.float32),
        compiler_params=pltpu.CompilerParams(
            dimension_semantics=("parallel",),
        ),
    )(embed, noise, maskf, stepsf, steprows, pe_pos, gamma3, beta3)


def kernel(X, steps, mask, emb_table, gamma, beta, noise):
    B, S = X.shape
    V, D = emb_table.shape
    N = B * S

    chunk = 128
    nw = 32
    idx = X.reshape(nw, N // (nw * chunk), chunk).astype(jnp.int32)
    steps_i = steps.astype(jnp.int32)
    pe_step = jnp.asarray(_sin_cos_encoding(D, MAX_STEPS))
    pe_pos = jnp.asarray(_sin_cos_encoding(D, PE_MAX)[:S][None])

    sc = _make_sc_gather(V, D, N, B, chunk=chunk)
    embed_flat, steprows = sc(emb_table, idx, pe_step, steps_i)

    embed = embed_flat.reshape(B, S, D)
    maskf = mask.astype(jnp.float32)
    stepsf = steps.astype(jnp.float32).reshape(B, 1, 1)
    steprows3 = steprows.reshape(B, 1, D)
    out = _epilogue(embed, noise, maskf, stepsf, steprows3,
                    pe_pos, gamma.reshape(1, 1, D), beta.reshape(1, 1, D), BB=32)
    return out


# speedup vs baseline: 4.2696x; 1.0218x over previous
import functools
import math

import jax
import jax.numpy as jnp
import numpy as np
from jax import lax
from jax.experimental import pallas as pl
from jax.experimental.pallas import tpu as pltpu
from jax.experimental.pallas import tpu_sc as plsc

MAX_STEPS = 1000
PE_MAX = 512


def _sin_cos_encoding(d_model, max_len):
    pe = np.zeros((max_len, d_model), dtype=np.float32)
    position = np.arange(0, max_len, dtype=np.float32)[:, None]
    div_term = np.exp(
        np.arange(0, d_model, 2, dtype=np.float32) * -(math.log(10000.0) / d_model)
    )
    pe[:, 0::2] = np.sin(position * div_term)
    pe[:, 1::2] = np.cos(position * div_term)
    return pe


def _make_sc_gather(V, D, N, B, chunk):
    info = plsc.get_sparse_core_info()
    nw = info.num_cores * info.num_subcores
    assert N % (nw * chunk) == 0 and B % nw == 0
    per_w = N // nw
    nchunks = per_w // chunk
    b_per_w = B // nw
    mesh = plsc.VectorSubcoreMesh(core_axis_name="c", subcore_axis_name="s")

    NBUF = 5
    assert nchunks % NBUF == 0 and nchunks >= NBUF

    @functools.partial(
        pl.kernel,
        mesh=mesh,
        out_type=(
            jax.ShapeDtypeStruct((N, D), jnp.float32),
            jax.ShapeDtypeStruct((B, D), jnp.float32),
        ),
        scratch_types=(
            [pltpu.VMEM((nchunks, chunk), jnp.int32)]
            + [pltpu.VMEM((chunk, D), jnp.float32) for _ in range(NBUF)]
            + [pltpu.VMEM((b_per_w,), jnp.int32),
               pltpu.VMEM((b_per_w, D), jnp.float32)]
            + [pltpu.SemaphoreType.DMA for _ in range(2 * NBUF + 1)]
        ),
    )
    def sc_gather(table_hbm, idx_hbm, pestep_hbm, steps_hbm,
                  emb_out, steprow_out, idx_v, *rest):
        rows = rest[:NBUF]
        sidx_v, srows_v = rest[NBUF:NBUF + 2]
        gsem = rest[NBUF + 2:2 * NBUF + 2]
        wsem = rest[2 * NBUF + 2:3 * NBUF + 2]
        sem = rest[3 * NBUF + 2]
        wid = lax.axis_index("s") * info.num_cores + lax.axis_index("c")

        def g_start(k, r):
            pltpu.async_copy(table_hbm.at[idx_v.at[k]], rows[r], gsem[r])

        def g_wait(k, r):
            pltpu.make_async_copy(table_hbm.at[idx_v.at[k]], rows[r], gsem[r]).wait()

        def w_start(k, r):
            base = wid * per_w + k * chunk
            pltpu.async_copy(rows[r], emb_out.at[pl.ds(base, chunk)], wsem[r])

        def w_wait(k, r):
            base = wid * per_w + k * chunk
            pltpu.make_async_copy(rows[r], emb_out.at[pl.ds(base, chunk)], wsem[r]).wait()

        pltpu.sync_copy(idx_hbm.at[wid], idx_v)
        g_start(0, 0)
        g_start(1, 1)
        g_start(2, 2)

        sbase = wid * b_per_w
        pltpu.sync_copy(steps_hbm.at[pl.ds(sbase, b_per_w)], sidx_v)
        pltpu.async_copy(pestep_hbm.at[sidx_v], srows_v, sem).wait()
        pltpu.sync_copy(srows_v, steprow_out.at[pl.ds(sbase, b_per_w)])

        def body(t, carry):
            for b in range(NBUF):
                k = NBUF * t + b
                g_wait(k, b)
                w_start(k, b)
                bj = (b + 3) % NBUF

                @pl.when(k + 3 >= NBUF)
                def _():
                    w_wait(k + 3 - NBUF, bj)

                @pl.when(k + 3 < nchunks)
                def _():
                    g_start(k + 3, bj)

            return carry

        lax.fori_loop(0, nchunks // NBUF, body, 0)
        for k in range(nchunks - (NBUF - 3), nchunks):
            w_wait(k, k % NBUF)

    return sc_gather


def _epilogue_body(emb_ref, noise_ref, mask_ref, steps_ref, steprow_ref,
                   pe_ref, gamma_ref, beta_ref, out_ref):
    e = emb_ref[...]
    nz = noise_ref[...]
    m = mask_ref[...][:, :, None]
    stf = steps_ref[...]
    ni = 1.0 - jnp.cos(jnp.pi * (1.0 - stf * (1.0 / MAX_STEPS)) * 0.5)
    x = e + m * ((ni - 1.0) * e + (1.0 - ni) * nz)
    x = x + pe_ref[...]
    x = x + m * steprow_ref[...]
    mean = jnp.mean(x, axis=-1, keepdims=True)
    xc = x - mean
    var = jnp.mean(xc * xc, axis=-1, keepdims=True)
    out_ref[...] = xc * lax.rsqrt(var + 1e-5) * gamma_ref[...] + beta_ref[...]


def _epilogue(embed, noise, maskf, stepsf, steprows, pe_pos, gamma3, beta3, BB):
    B, S, D = embed.shape
    grid = (B // BB,)
    bspec = pl.BlockSpec((BB, S, D), lambda i: (i, 0, 0))
    return pl.pallas_call(
        _epilogue_body,
        grid=grid,
        in_specs=[
            bspec,
            bspec,
            pl.BlockSpec((BB, S), lambda i: (i, 0)),
            pl.BlockSpec((BB, 1, 1), lambda i: (i, 0, 0)),
            pl.BlockSpec((BB, 1, D), lambda i: (i, 0, 0)),
            pl.BlockSpec((1, S, D), lambda i: (0, 0, 0)),
            pl.BlockSpec((1, 1, D), lambda i: (0, 0, 0)),
            pl.BlockSpec((1, 1, D), lambda i: (0, 0, 0)),
        ],
        out_specs=bspec,
        out_shape=jax.ShapeDtypeStruct((B, S, D), jnp.float32),
        compiler_params=pltpu.CompilerParams(
            dimension_semantics=("parallel",),
        ),
    )(embed, noise, maskf, stepsf, steprows, pe_pos, gamma3, beta3)


def kernel(X, steps, mask, emb_table, gamma, beta, noise):
    B, S = X.shape
    V, D = emb_table.shape
    N = B * S

    chunk = 128
    nw = 32
    idx = X.reshape(nw, N // (nw * chunk), chunk).astype(jnp.int32)
    steps_i = steps.astype(jnp.int32)
    pe_step = jnp.asarray(_sin_cos_encoding(D, MAX_STEPS))
    pe_pos = jnp.asarray(_sin_cos_encoding(D, PE_MAX)[:S][None])

    sc = _make_sc_gather(V, D, N, B, chunk=chunk)
    embed_flat, steprows = sc(emb_table, idx, pe_step, steps_i)

    embed = embed_flat.reshape(B, S, D)
    maskf = mask.astype(jnp.float32)
    stepsf = steps.astype(jnp.float32).reshape(B, 1, 1)
    steprows3 = steprows.reshape(B, 1, D)
    out = _epilogue(embed, noise, maskf, stepsf, steprows3,
                    pe_pos, gamma.reshape(1, 1, D), beta.reshape(1, 1, D), BB=64)
    return out
